# baseline (speedup 1.0000x reference)
import jax
import jax.numpy as jnp
from jax import lax
from jax.experimental import pallas as pl
from jax.experimental.pallas import tpu as pltpu
from jax.experimental.pallas import tpu_sc as plsc

HEADS = 8
DIM_HEAD = 32
EPS = 1e-5
TB = 256


def _ln_lanes(x, g, b):
    mu = jnp.mean(x, axis=1, keepdims=True)
    xc = x - mu
    var = jnp.mean(xc * xc, axis=1, keepdims=True)
    return xc / jnp.sqrt(var + EPS) * g + b


def _stage_a1(qsT_ref, ctxT_ref, wkvT_ref, wqT_ref, gq_ref, bq_ref, gc_ref,
              bc_ref, idx_ref, qh_ref, kh_ref, vh_ref, qsel_ref):
    ctxn = _ln_lanes(ctxT_ref[0].T, gc_ref[:], bc_ref[:])
    qsn = _ln_lanes(qsT_ref[0].T, gq_ref[:], bq_ref[:])
    bf = jnp.bfloat16
    kv = lax.dot_general(ctxn.astype(bf), wkvT_ref[:].astype(bf),
                         (((1,), (0,)), ((), ())),
                         preferred_element_type=jnp.float32)
    q = lax.dot_general(qsn.astype(bf), wqT_ref[:].astype(bf),
                        (((1,), (0,)), ((), ())),
                        preferred_element_type=jnp.float32)
    inner = q.shape[1]
    for h in range(HEADS):
        lo = h * DIM_HEAD
        qh = q[:, lo:lo + DIM_HEAD]
        kh = kv[:, lo:lo + DIM_HEAD]
        qn = qh / jnp.maximum(jnp.sqrt(jnp.sum(qh * qh, axis=1, keepdims=True)), 1e-12)
        kn = kh / jnp.maximum(jnp.sqrt(jnp.sum(kh * kh, axis=1, keepdims=True)), 1e-12)
        qh_ref[0, h] = qn
        kh_ref[0, h] = kn
        vh_ref[0, h] = kv[:, inner + lo:inner + lo + DIM_HEAD]
        idxh = idx_ref[0, h, 0]
        S = idxh.shape[0]
        L = qn.shape[0]
        oh = (lax.broadcasted_iota(jnp.int32, (S, L), 1) == idxh[:, None]
              ).astype(jnp.float32)
        Qs = lax.dot_general(oh, qn, (((1,), (0,)), ((), ())),
                             preferred_element_type=jnp.float32,
                             precision=lax.Precision.HIGHEST)
        qsel_ref[0, h] = Qs.T


def _stage_a3(kh_ref, qsT_ref, mind_ref):
    kb = kh_ref[0, 0]
    QsT = qsT_ref[0, 0]
    d3 = jnp.abs(kb[:, :, None] - QsT[None, :, :])
    acc = jnp.sum(d3, axis=1)
    mind_ref[0, 0] = jnp.min(acc, axis=1, keepdims=True)


def _stage_c(qh_ref, ksel_ref, vsel_ref, woutT_ref, go_ref, bo_ref, gam_ref,
             qsrcT_ref, y_ref):
    outs = []
    for h in range(HEADS):
        qn = qh_ref[0, h]
        ks = ksel_ref[0, h]
        vs = vsel_ref[0, h]
        logits = lax.dot_general(qn, ks, (((1,), (1,)), ((), ())),
                                 preferred_element_type=jnp.float32)
        m = jnp.max(logits, axis=1, keepdims=True)
        p = jnp.exp(logits - m)
        a = p / jnp.sum(p, axis=1, keepdims=True)
        outs.append(lax.dot_general(a, vs, (((1,), (0,)), ((), ())),
                                    preferred_element_type=jnp.float32))
    o = jnp.concatenate(outs, axis=1)
    y = lax.dot_general(o, woutT_ref[:], (((1,), (0,)), ((), ())),
                        preferred_element_type=jnp.float32)
    y = _ln_lanes(y, go_ref[:], bo_ref[:])
    y_ref[0] = gam_ref[0, 0] * y.T + qsrcT_ref[0]


def _make_sc_select_gather(L, K):
    nch = L // 16

    def body(mind_hbm, kt_hbm, vt_hbm, ko_hbm, vo_hbm, mind_v, idx_v, rows_v, sem):
        r = lax.axis_index("s")
        c = lax.axis_index("c")
        pltpu.sync_copy(mind_hbm.at[r], mind_v)

        def count_lt(trial):
            acc = jnp.zeros((16,), jnp.int32)
            for i in range(nch):
                u = plsc.bitcast(mind_v[pl.ds(i * 16, 16)], jnp.int32)
                acc = acc + jnp.where(u < trial, 1, 0).astype(jnp.int32)
            return jnp.sum(acc)

        def bit_step(j, res):
            trial = res | lax.shift_left(jnp.int32(1), jnp.int32(30) - j)
            return jnp.where(count_lt(trial) < K, trial, res)

        thr = lax.fori_loop(0, 31, bit_step, jnp.int32(0))
        n_less = count_lt(thr)

        base = jnp.int32(0)
        need = jnp.int32(K) - n_less
        row_off = (r * L).astype(jnp.int32)
        for i in range(nch):
            u = plsc.bitcast(mind_v[pl.ds(i * 16, 16)], jnp.int32)
            less = u < thr
            eq = u == thr
            eq_i = jnp.where(eq, 1, 0).astype(jnp.int32)
            eq_pos = plsc.cumsum(eq_i) - eq_i
            sel = less | (eq & (eq_pos < need))
            sel_i = jnp.where(sel, 1, 0).astype(jnp.int32)
            pos = base + plsc.cumsum(sel_i) - sel_i
            lanes = jnp.int32(i * 16) + lax.iota(jnp.int32, 16) + row_off
            plsc.store_scatter(idx_v, [pos], lanes, mask=sel)
            base = base + jnp.sum(sel_i)
            need = need - jnp.sum(jnp.where(eq & (eq_pos < need), 1, 0).astype(jnp.int32))

        @pl.when(c == 0)
        def _():
            pltpu.async_copy(kt_hbm.at[idx_v], rows_v, sem).wait()
            pltpu.sync_copy(rows_v, ko_hbm.at[r])

        @pl.when(c == 1)
        def _():
            pltpu.async_copy(vt_hbm.at[idx_v], rows_v, sem).wait()
            pltpu.sync_copy(rows_v, vo_hbm.at[r])

    return body


def kernel(query_source, context, w_kv, w_q, w_out, g_ctx, b_ctx, g_q, b_q,
           g_out, b_out, gamma):
    b, C, D, H, W = query_source.shape
    L = D * H * W
    BH = b * HEADS
    K = L // HEADS
    inner = HEADS * DIM_HEAD

    qsT = query_source.reshape(b, C, L)
    ctxT = context.reshape(b, C, L)
    wkvT = w_kv.T
    wqT = w_q.T
    woutT = w_out.T
    gc = g_ctx.reshape(1, C)
    bc = b_ctx.reshape(1, C)
    gq = g_q.reshape(1, C)
    bq = b_q.reshape(1, C)
    go = g_out.reshape(1, C)
    bo = b_out.reshape(1, C)
    gam = gamma.reshape(1, 1)

    sel_key = jax.random.key(1234)
    idx = jax.random.randint(sel_key, (BH, min(K, L)), 0, L).astype(jnp.int32)
    idx = idx.reshape(b, HEADS, 1, K)

    arb = lambda n: pltpu.CompilerParams(dimension_semantics=("arbitrary",) * n)

    qh, kh, vh, qsel_t = pl.pallas_call(
        _stage_a1,
        grid=(b,),
        in_specs=[
            pl.BlockSpec((1, C, L), lambda i: (i, 0, 0)),
            pl.BlockSpec((1, C, L), lambda i: (i, 0, 0)),
            pl.BlockSpec((C, 2 * inner), lambda i: (0, 0)),
            pl.BlockSpec((C, inner), lambda i: (0, 0)),
            pl.BlockSpec((1, C), lambda i: (0, 0)),
            pl.BlockSpec((1, C), lambda i: (0, 0)),
            pl.BlockSpec((1, C), lambda i: (0, 0)),
            pl.BlockSpec((1, C), lambda i: (0, 0)),
            pl.BlockSpec((1, HEADS, 1, K), lambda i: (i, 0, 0, 0)),
        ],
        out_specs=[
            pl.BlockSpec((1, HEADS, L, DIM_HEAD), lambda i: (i, 0, 0, 0)),
            pl.BlockSpec((1, HEADS, L, DIM_HEAD), lambda i: (i, 0, 0, 0)),
            pl.BlockSpec((1, HEADS, L, DIM_HEAD), lambda i: (i, 0, 0, 0)),
            pl.BlockSpec((1, HEADS, DIM_HEAD, K), lambda i: (i, 0, 0, 0)),
        ],
        out_shape=[
            jax.ShapeDtypeStruct((b, HEADS, L, DIM_HEAD), jnp.float32),
            jax.ShapeDtypeStruct((b, HEADS, L, DIM_HEAD), jnp.float32),
            jax.ShapeDtypeStruct((b, HEADS, L, DIM_HEAD), jnp.float32),
            jax.ShapeDtypeStruct((b, HEADS, DIM_HEAD, K), jnp.float32),
        ],
        compiler_params=arb(1),
    )(qsT, ctxT, wkvT, wqT, gq, bq, gc, bc, idx)

    nblk = L // TB
    mind = pl.pallas_call(
        _stage_a3,
        grid=(b, HEADS, nblk),
        in_specs=[
            pl.BlockSpec((1, 1, TB, DIM_HEAD), lambda i, j, t: (i, j, t, 0)),
            pl.BlockSpec((1, 1, DIM_HEAD, K), lambda i, j, t: (i, j, 0, 0)),
        ],
        out_specs=pl.BlockSpec((1, 1, TB, 1), lambda i, j, t: (i, j, t, 0)),
        out_shape=jax.ShapeDtypeStruct((b, HEADS, L, 1), jnp.float32),
        compiler_params=arb(3),
    )(kh, qsel_t)

    mind16 = mind.reshape(BH, L)
    kt = kh.reshape(BH * L, DIM_HEAD)
    vt = vh.reshape(BH * L, DIM_HEAD)

    mesh = plsc.VectorSubcoreMesh(core_axis_name="c", subcore_axis_name="s")
    ko, vo = pl.kernel(
        _make_sc_select_gather(L, K),
        out_type=(
            jax.ShapeDtypeStruct((BH, K, DIM_HEAD), jnp.float32),
            jax.ShapeDtypeStruct((BH, K, DIM_HEAD), jnp.float32),
        ),
        mesh=mesh,
        scratch_types=[
            pltpu.VMEM((L,), jnp.float32),
            pltpu.VMEM((K,), jnp.int32),
            pltpu.VMEM((K, DIM_HEAD), jnp.float32),
            pltpu.SemaphoreType.DMA,
        ],
        compiler_params=pltpu.CompilerParams(
            needs_layout_passes=False, use_tc_tiling_on_sc=False),
    )(mind16, kt, vt)

    ksel = ko.reshape(b, HEADS, K, DIM_HEAD)
    vsel = vo.reshape(b, HEADS, K, DIM_HEAD)

    yT = pl.pallas_call(
        _stage_c,
        grid=(b,),
        in_specs=[
            pl.BlockSpec((1, HEADS, L, DIM_HEAD), lambda i: (i, 0, 0, 0)),
            pl.BlockSpec((1, HEADS, K, DIM_HEAD), lambda i: (i, 0, 0, 0)),
            pl.BlockSpec((1, HEADS, K, DIM_HEAD), lambda i: (i, 0, 0, 0)),
            pl.BlockSpec((C, inner), lambda i: (0, 0)),
            pl.BlockSpec((1, C), lambda i: (0, 0)),
            pl.BlockSpec((1, C), lambda i: (0, 0)),
            pl.BlockSpec((1, 1), lambda i: (0, 0)),
            pl.BlockSpec((1, C, L), lambda i: (i, 0, 0)),
        ],
        out_specs=pl.BlockSpec((1, C, L), lambda i: (i, 0, 0)),
        out_shape=jax.ShapeDtypeStruct((b, C, L), jnp.float32),
        compiler_params=arb(1),
    )(qh, ksel, vsel, woutT, go, bo, gam, qsT)

    return yT.reshape(b, C, D, H, W)


# speedup vs baseline: 1.0307x; 1.0307x over previous
import jax
import jax.numpy as jnp
from jax import lax
from jax.experimental import pallas as pl
from jax.experimental.pallas import tpu as pltpu
from jax.experimental.pallas import tpu_sc as plsc

HEADS = 8
DIM_HEAD = 32
EPS = 1e-5
TB = 256


def _ln_lanes(x, g, b):
    mu = jnp.mean(x, axis=1, keepdims=True)
    xc = x - mu
    var = jnp.mean(xc * xc, axis=1, keepdims=True)
    return xc / jnp.sqrt(var + EPS) * g + b


def _stage_a1(qsT_ref, ctxT_ref, wkvT_ref, wqT_ref, gq_ref, bq_ref, gc_ref,
              bc_ref, idx_ref, qh_ref, kh_ref, vh_ref, qsel_ref):
    ctxn = _ln_lanes(ctxT_ref[0], gc_ref[:], bc_ref[:])
    qsn = _ln_lanes(qsT_ref[0], gq_ref[:], bq_ref[:])
    bf = jnp.bfloat16
    kv = lax.dot_general(ctxn.astype(bf), wkvT_ref[:].astype(bf),
                         (((1,), (0,)), ((), ())),
                         preferred_element_type=jnp.float32)
    q = lax.dot_general(qsn.astype(bf), wqT_ref[:].astype(bf),
                        (((1,), (0,)), ((), ())),
                        preferred_element_type=jnp.float32)
    inner = q.shape[1]
    for h in range(HEADS):
        lo = h * DIM_HEAD
        qh = q[:, lo:lo + DIM_HEAD]
        kh = kv[:, lo:lo + DIM_HEAD]
        qn = qh / jnp.maximum(jnp.sqrt(jnp.sum(qh * qh, axis=1, keepdims=True)), 1e-12)
        kn = kh / jnp.maximum(jnp.sqrt(jnp.sum(kh * kh, axis=1, keepdims=True)), 1e-12)
        qh_ref[0, h] = qn
        kh_ref[0, h] = kn
        vh_ref[0, h] = kv[:, inner + lo:inner + lo + DIM_HEAD]
        idxh = idx_ref[0, h, 0]
        S = idxh.shape[0]
        L = qn.shape[0]
        oh = (lax.broadcasted_iota(jnp.int32, (S, L), 1) == idxh[:, None]
              ).astype(jnp.float32)
        Qs = lax.dot_general(oh, qn, (((1,), (0,)), ((), ())),
                             preferred_element_type=jnp.float32,
                             precision=lax.Precision.HIGHEST)
        qsel_ref[0, h] = Qs.T


def _stage_a3(kh_ref, qsT_ref, mind_ref):
    kb = kh_ref[0, 0]
    QsT = qsT_ref[0, 0]
    acc = None
    for d in range(DIM_HEAD):
        term = jnp.abs(kb[:, d:d + 1] - QsT[d:d + 1, :])
        acc = term if acc is None else acc + term
    mind_ref[0, 0] = jnp.min(acc, axis=1, keepdims=True)


def _stage_c(qh_ref, ksel_ref, vsel_ref, woutT_ref, go_ref, bo_ref, gam_ref,
             qsrcT_ref, y_ref):
    outs = []
    for h in range(HEADS):
        qn = qh_ref[0, h]
        ks = ksel_ref[0, h]
        vs = vsel_ref[0, h]
        logits = lax.dot_general(qn, ks, (((1,), (1,)), ((), ())),
                                 preferred_element_type=jnp.float32)
        m = jnp.max(logits, axis=1, keepdims=True)
        p = jnp.exp(logits - m)
        a = p / jnp.sum(p, axis=1, keepdims=True)
        outs.append(lax.dot_general(a, vs, (((1,), (0,)), ((), ())),
                                    preferred_element_type=jnp.float32))
    o = jnp.concatenate(outs, axis=1)
    y = lax.dot_general(o, woutT_ref[:], (((1,), (0,)), ((), ())),
                        preferred_element_type=jnp.float32)
    y = _ln_lanes(y, go_ref[:], bo_ref[:])
    y_ref[0] = gam_ref[0, 0] * y + qsrcT_ref[0]


def _make_sc_select_gather(L, K):
    nch = L // 16

    def body(mind_hbm, kt_hbm, vt_hbm, ko_hbm, vo_hbm, mind_v, idx_v, rows_v, sem):
        r = lax.axis_index("s")
        c = lax.axis_index("c")
        pltpu.sync_copy(mind_hbm.at[r], mind_v)

        def count_lt(trial):
            acc = jnp.zeros((16,), jnp.int32)
            for i in range(nch):
                u = plsc.bitcast(mind_v[pl.ds(i * 16, 16)], jnp.int32)
                acc = acc + jnp.where(u < trial, 1, 0).astype(jnp.int32)
            return jnp.sum(acc)

        def bit_step(j, res):
            trial = res | lax.shift_left(jnp.int32(1), jnp.int32(30) - j)
            return jnp.where(count_lt(trial) < K, trial, res)

        thr = lax.fori_loop(0, 31, bit_step, jnp.int32(0))
        n_less = count_lt(thr)

        base = jnp.int32(0)
        need = jnp.int32(K) - n_less
        row_off = (r * L).astype(jnp.int32)
        for i in range(nch):
            u = plsc.bitcast(mind_v[pl.ds(i * 16, 16)], jnp.int32)
            less = u < thr
            eq = u == thr
            eq_i = jnp.where(eq, 1, 0).astype(jnp.int32)
            eq_pos = plsc.cumsum(eq_i) - eq_i
            sel = less | (eq & (eq_pos < need))
            sel_i = jnp.where(sel, 1, 0).astype(jnp.int32)
            pos = base + plsc.cumsum(sel_i) - sel_i
            lanes = jnp.int32(i * 16) + lax.iota(jnp.int32, 16) + row_off
            plsc.store_scatter(idx_v, [pos], lanes, mask=sel)
            base = base + jnp.sum(sel_i)
            need = need - jnp.sum(jnp.where(eq & (eq_pos < need), 1, 0).astype(jnp.int32))

        @pl.when(c == 0)
        def _():
            pltpu.async_copy(kt_hbm.at[idx_v], rows_v, sem).wait()
            pltpu.sync_copy(rows_v, ko_hbm.at[r])

        @pl.when(c == 1)
        def _():
            pltpu.async_copy(vt_hbm.at[idx_v], rows_v, sem).wait()
            pltpu.sync_copy(rows_v, vo_hbm.at[r])

    return body


def kernel(query_source, context, w_kv, w_q, w_out, g_ctx, b_ctx, g_q, b_q,
           g_out, b_out, gamma):
    b, C, D, H, W = query_source.shape
    L = D * H * W
    BH = b * HEADS
    K = L // HEADS
    inner = HEADS * DIM_HEAD

    qsT = query_source.reshape(b, C, L).transpose(0, 2, 1)
    ctxT = context.reshape(b, C, L).transpose(0, 2, 1)
    wkvT = w_kv.T
    wqT = w_q.T
    woutT = w_out.T
    gc = g_ctx.reshape(1, C)
    bc = b_ctx.reshape(1, C)
    gq = g_q.reshape(1, C)
    bq = b_q.reshape(1, C)
    go = g_out.reshape(1, C)
    bo = b_out.reshape(1, C)
    gam = gamma.reshape(1, 1)

    idx = jax.random.randint(jax.random.key(1234), (BH, min(K, L)), 0, L
                             ).astype(jnp.int32).reshape(b, HEADS, 1, K)

    arb = lambda n: pltpu.CompilerParams(dimension_semantics=("arbitrary",) * n)

    qh, kh, vh, qsel_t = pl.pallas_call(
        _stage_a1,
        grid=(b,),
        in_specs=[
            pl.BlockSpec((1, L, C), lambda i: (i, 0, 0)),
            pl.BlockSpec((1, L, C), lambda i: (i, 0, 0)),
            pl.BlockSpec((C, 2 * inner), lambda i: (0, 0)),
            pl.BlockSpec((C, inner), lambda i: (0, 0)),
            pl.BlockSpec((1, C), lambda i: (0, 0)),
            pl.BlockSpec((1, C), lambda i: (0, 0)),
            pl.BlockSpec((1, C), lambda i: (0, 0)),
            pl.BlockSpec((1, C), lambda i: (0, 0)),
            pl.BlockSpec((1, HEADS, 1, K), lambda i: (i, 0, 0, 0)),
        ],
        out_specs=[
            pl.BlockSpec((1, HEADS, L, DIM_HEAD), lambda i: (i, 0, 0, 0)),
            pl.BlockSpec((1, HEADS, L, DIM_HEAD), lambda i: (i, 0, 0, 0)),
            pl.BlockSpec((1, HEADS, L, DIM_HEAD), lambda i: (i, 0, 0, 0)),
            pl.BlockSpec((1, HEADS, DIM_HEAD, K), lambda i: (i, 0, 0, 0)),
        ],
        out_shape=[
            jax.ShapeDtypeStruct((b, HEADS, L, DIM_HEAD), jnp.float32),
            jax.ShapeDtypeStruct((b, HEADS, L, DIM_HEAD), jnp.float32),
            jax.ShapeDtypeStruct((b, HEADS, L, DIM_HEAD), jnp.float32),
            jax.ShapeDtypeStruct((b, HEADS, DIM_HEAD, K), jnp.float32),
        ],
        compiler_params=arb(1),
    )(qsT, ctxT, wkvT, wqT, gq, bq, gc, bc, idx)

    nblk = L // TB
    mind = pl.pallas_call(
        _stage_a3,
        grid=(b, HEADS, nblk),
        in_specs=[
            pl.BlockSpec((1, 1, TB, DIM_HEAD), lambda i, j, t: (i, j, t, 0)),
            pl.BlockSpec((1, 1, DIM_HEAD, K), lambda i, j, t: (i, j, 0, 0)),
        ],
        out_specs=pl.BlockSpec((1, 1, TB, 1), lambda i, j, t: (i, j, t, 0)),
        out_shape=jax.ShapeDtypeStruct((b, HEADS, L, 1), jnp.float32),
        compiler_params=arb(3),
    )(kh, qsel_t)

    mind16 = mind.reshape(BH, L)
    kt = kh.reshape(BH * L, DIM_HEAD)
    vt = vh.reshape(BH * L, DIM_HEAD)

    mesh = plsc.VectorSubcoreMesh(core_axis_name="c", subcore_axis_name="s")
    ko, vo = pl.kernel(
        _make_sc_select_gather(L, K),
        out_type=(
            jax.ShapeDtypeStruct((BH, K, DIM_HEAD), jnp.float32),
            jax.ShapeDtypeStruct((BH, K, DIM_HEAD), jnp.float32),
        ),
        mesh=mesh,
        scratch_types=[
            pltpu.VMEM((L,), jnp.float32),
            pltpu.VMEM((K,), jnp.int32),
            pltpu.VMEM((K, DIM_HEAD), jnp.float32),
            pltpu.SemaphoreType.DMA,
        ],
        compiler_params=pltpu.CompilerParams(
            needs_layout_passes=False, use_tc_tiling_on_sc=False),
    )(mind16, kt, vt)

    ksel = ko.reshape(b, HEADS, K, DIM_HEAD)
    vsel = vo.reshape(b, HEADS, K, DIM_HEAD)

    yT = pl.pallas_call(
        _stage_c,
        grid=(b,),
        in_specs=[
            pl.BlockSpec((1, HEADS, L, DIM_HEAD), lambda i: (i, 0, 0, 0)),
            pl.BlockSpec((1, HEADS, K, DIM_HEAD), lambda i: (i, 0, 0, 0)),
            pl.BlockSpec((1, HEADS, K, DIM_HEAD), lambda i: (i, 0, 0, 0)),
            pl.BlockSpec((C, inner), lambda i: (0, 0)),
            pl.BlockSpec((1, C), lambda i: (0, 0)),
            pl.BlockSpec((1, C), lambda i: (0, 0)),
            pl.BlockSpec((1, 1), lambda i: (0, 0)),
            pl.BlockSpec((1, L, C), lambda i: (i, 0, 0)),
        ],
        out_specs=pl.BlockSpec((1, L, C), lambda i: (i, 0, 0)),
        out_shape=jax.ShapeDtypeStruct((b, L, C), jnp.float32),
        compiler_params=arb(1),
    )(qh, ksel, vsel, woutT, go, bo, gam, qsT)

    return yT.transpose(0, 2, 1).reshape(b, C, D, H, W)


# speedup vs baseline: 1.0548x; 1.0234x over previous
import jax
import jax.numpy as jnp
from jax import lax
from jax.experimental import pallas as pl
from jax.experimental.pallas import tpu as pltpu
from jax.experimental.pallas import tpu_sc as plsc

HEADS = 8
DIM_HEAD = 32
EPS = 1e-5
TB = 256


def _ln_lanes(x, g, b):
    mu = jnp.mean(x, axis=1, keepdims=True)
    xc = x - mu
    var = jnp.mean(xc * xc, axis=1, keepdims=True)
    return xc / jnp.sqrt(var + EPS) * g + b


def _stage_a1(qsT_ref, ctxT_ref, wkvT_ref, wqT_ref, gq_ref, bq_ref, gc_ref,
              bc_ref, idx_ref, qh_ref, kh_ref, vh_ref, qsel_ref):
    ctxn = _ln_lanes(ctxT_ref[0], gc_ref[:], bc_ref[:])
    qsn = _ln_lanes(qsT_ref[0], gq_ref[:], bq_ref[:])
    bf = jnp.bfloat16
    kv = lax.dot_general(ctxn.astype(bf), wkvT_ref[:].astype(bf),
                         (((1,), (0,)), ((), ())),
                         preferred_element_type=jnp.float32)
    q = lax.dot_general(qsn.astype(bf), wqT_ref[:].astype(bf),
                        (((1,), (0,)), ((), ())),
                        preferred_element_type=jnp.float32)
    inner = q.shape[1]
    for h in range(HEADS):
        lo = h * DIM_HEAD
        qh = q[:, lo:lo + DIM_HEAD]
        kh = kv[:, lo:lo + DIM_HEAD]
        qn = qh / jnp.maximum(jnp.sqrt(jnp.sum(qh * qh, axis=1, keepdims=True)), 1e-12)
        kn = kh / jnp.maximum(jnp.sqrt(jnp.sum(kh * kh, axis=1, keepdims=True)), 1e-12)
        qh_ref[0, h] = qn
        kh_ref[0, h] = kn
        vh_ref[0, h] = kv[:, inner + lo:inner + lo + DIM_HEAD]
        idxh = idx_ref[0, h, 0]
        S = idxh.shape[0]
        L = qn.shape[0]
        oh = (lax.broadcasted_iota(jnp.int32, (S, L), 1) == idxh[:, None]
              ).astype(jnp.float32)
        Qs = lax.dot_general(oh, qn, (((1,), (0,)), ((), ())),
                             preferred_element_type=jnp.float32,
                             precision=lax.Precision.HIGHEST)
        qsel_ref[0, h] = Qs.T


def _stage_a3(kh_ref, qsT_ref, mind_ref):
    kb = kh_ref[0, 0]
    QsT = qsT_ref[0, 0]
    d3 = jnp.abs(kb[:, :, None] - QsT[None, :, :])
    acc = jnp.sum(d3, axis=1)
    mind_ref[0, 0] = jnp.min(acc, axis=1, keepdims=True)


def _stage_c(qh_ref, ksel_ref, vsel_ref, woutT_ref, go_ref, bo_ref, gam_ref,
             qsrcT_ref, y_ref):
    outs = []
    for h in range(HEADS):
        qn = qh_ref[0, h]
        ks = ksel_ref[0, h]
        vs = vsel_ref[0, h]
        logits = lax.dot_general(qn, ks, (((1,), (1,)), ((), ())),
                                 preferred_element_type=jnp.float32)
        m = jnp.max(logits, axis=1, keepdims=True)
        p = jnp.exp(logits - m)
        a = p / jnp.sum(p, axis=1, keepdims=True)
        outs.append(lax.dot_general(a, vs, (((1,), (0,)), ((), ())),
                                    preferred_element_type=jnp.float32))
    o = jnp.concatenate(outs, axis=1)
    y = lax.dot_general(o, woutT_ref[:], (((1,), (0,)), ((), ())),
                        preferred_element_type=jnp.float32)
    y = _ln_lanes(y, go_ref[:], bo_ref[:])
    y_ref[0] = gam_ref[0, 0] * y + qsrcT_ref[0]


def _make_sc_select_gather(L, K):
    nch = L // 16

    def body(mind_hbm, kt_hbm, vt_hbm, ko_hbm, vo_hbm, mind_v, idx_v, rows_v, sem):
        r = lax.axis_index("s")
        c = lax.axis_index("c")
        pltpu.sync_copy(mind_hbm.at[r], mind_v)

        def count_lt(trial):
            acc = jnp.zeros((16,), jnp.int32)
            for i in range(nch):
                u = plsc.bitcast(mind_v[pl.ds(i * 16, 16)], jnp.int32)
                acc = acc + jnp.where(u < trial, 1, 0).astype(jnp.int32)
            return jnp.sum(acc)

        def bit_step(j, res):
            trial = res | lax.shift_left(jnp.int32(1), jnp.int32(30) - j)
            return jnp.where(count_lt(trial) < K, trial, res)

        thr = lax.fori_loop(0, 31, bit_step, jnp.int32(0))
        n_less = count_lt(thr)

        base = jnp.int32(0)
        need = jnp.int32(K) - n_less
        row_off = (r * L).astype(jnp.int32)
        for i in range(nch):
            u = plsc.bitcast(mind_v[pl.ds(i * 16, 16)], jnp.int32)
            less = u < thr
            eq = u == thr
            eq_i = jnp.where(eq, 1, 0).astype(jnp.int32)
            eq_pos = plsc.cumsum(eq_i) - eq_i
            sel = less | (eq & (eq_pos < need))
            sel_i = jnp.where(sel, 1, 0).astype(jnp.int32)
            pos = base + plsc.cumsum(sel_i) - sel_i
            lanes = jnp.int32(i * 16) + lax.iota(jnp.int32, 16) + row_off
            plsc.store_scatter(idx_v, [pos], lanes, mask=sel)
            base = base + jnp.sum(sel_i)
            need = need - jnp.sum(jnp.where(eq & (eq_pos < need), 1, 0).astype(jnp.int32))

        @pl.when(c == 0)
        def _():
            pltpu.async_copy(kt_hbm.at[idx_v], rows_v, sem).wait()
            pltpu.sync_copy(rows_v, ko_hbm.at[r])

        @pl.when(c == 1)
        def _():
            pltpu.async_copy(vt_hbm.at[idx_v], rows_v, sem).wait()
            pltpu.sync_copy(rows_v, vo_hbm.at[r])

    return body


def kernel(query_source, context, w_kv, w_q, w_out, g_ctx, b_ctx, g_q, b_q,
           g_out, b_out, gamma):
    b, C, D, H, W = query_source.shape
    L = D * H * W
    BH = b * HEADS
    K = L // HEADS
    inner = HEADS * DIM_HEAD

    qsT = query_source.reshape(b, C, L).transpose(0, 2, 1)
    ctxT = context.reshape(b, C, L).transpose(0, 2, 1)
    wkvT = w_kv.T
    wqT = w_q.T
    woutT = w_out.T
    gc = g_ctx.reshape(1, C)
    bc = b_ctx.reshape(1, C)
    gq = g_q.reshape(1, C)
    bq = b_q.reshape(1, C)
    go = g_out.reshape(1, C)
    bo = b_out.reshape(1, C)
    gam = gamma.reshape(1, 1)

    idx = jax.random.randint(jax.random.key(1234), (BH, min(K, L)), 0, L
                             ).astype(jnp.int32).reshape(b, HEADS, 1, K)

    arb = lambda n: pltpu.CompilerParams(dimension_semantics=("arbitrary",) * n)

    qh, kh, vh, qsel_t = pl.pallas_call(
        _stage_a1,
        grid=(b,),
        in_specs=[
            pl.BlockSpec((1, L, C), lambda i: (i, 0, 0)),
            pl.BlockSpec((1, L, C), lambda i: (i, 0, 0)),
            pl.BlockSpec((C, 2 * inner), lambda i: (0, 0)),
            pl.BlockSpec((C, inner), lambda i: (0, 0)),
            pl.BlockSpec((1, C), lambda i: (0, 0)),
            pl.BlockSpec((1, C), lambda i: (0, 0)),
            pl.BlockSpec((1, C), lambda i: (0, 0)),
            pl.BlockSpec((1, C), lambda i: (0, 0)),
            pl.BlockSpec((1, HEADS, 1, K), lambda i: (i, 0, 0, 0)),
        ],
        out_specs=[
            pl.BlockSpec((1, HEADS, L, DIM_HEAD), lambda i: (i, 0, 0, 0)),
            pl.BlockSpec((1, HEADS, L, DIM_HEAD), lambda i: (i, 0, 0, 0)),
            pl.BlockSpec((1, HEADS, L, DIM_HEAD), lambda i: (i, 0, 0, 0)),
            pl.BlockSpec((1, HEADS, DIM_HEAD, K), lambda i: (i, 0, 0, 0)),
        ],
        out_shape=[
            jax.ShapeDtypeStruct((b, HEADS, L, DIM_HEAD), jnp.float32),
            jax.ShapeDtypeStruct((b, HEADS, L, DIM_HEAD), jnp.float32),
            jax.ShapeDtypeStruct((b, HEADS, L, DIM_HEAD), jnp.float32),
            jax.ShapeDtypeStruct((b, HEADS, DIM_HEAD, K), jnp.float32),
        ],
        compiler_params=arb(1),
    )(qsT, ctxT, wkvT, wqT, gq, bq, gc, bc, idx)

    nblk = L // TB
    mind = pl.pallas_call(
        _stage_a3,
        grid=(b, HEADS, nblk),
        in_specs=[
            pl.BlockSpec((1, 1, TB, DIM_HEAD), lambda i, j, t: (i, j, t, 0)),
            pl.BlockSpec((1, 1, DIM_HEAD, K), lambda i, j, t: (i, j, 0, 0)),
        ],
        out_specs=pl.BlockSpec((1, 1, TB, 1), lambda i, j, t: (i, j, t, 0)),
        out_shape=jax.ShapeDtypeStruct((b, HEADS, L, 1), jnp.float32),
        compiler_params=arb(3),
    )(kh, qsel_t)

    mind16 = mind.reshape(BH, L)
    kt = kh.reshape(BH * L, DIM_HEAD)
    vt = vh.reshape(BH * L, DIM_HEAD)

    mesh = plsc.VectorSubcoreMesh(core_axis_name="c", subcore_axis_name="s")
    ko, vo = pl.kernel(
        _make_sc_select_gather(L, K),
        out_type=(
            jax.ShapeDtypeStruct((BH, K, DIM_HEAD), jnp.float32),
            jax.ShapeDtypeStruct((BH, K, DIM_HEAD), jnp.float32),
        ),
        mesh=mesh,
        scratch_types=[
            pltpu.VMEM((L,), jnp.float32),
            pltpu.VMEM((K,), jnp.int32),
            pltpu.VMEM((K, DIM_HEAD), jnp.float32),
            pltpu.SemaphoreType.DMA,
        ],
        compiler_params=pltpu.CompilerParams(
            needs_layout_passes=False, use_tc_tiling_on_sc=False),
    )(mind16, kt, vt)

    ksel = ko.reshape(b, HEADS, K, DIM_HEAD)
    vsel = vo.reshape(b, HEADS, K, DIM_HEAD)

    yT = pl.pallas_call(
        _stage_c,
        grid=(b,),
        in_specs=[
            pl.BlockSpec((1, HEADS, L, DIM_HEAD), lambda i: (i, 0, 0, 0)),
            pl.BlockSpec((1, HEADS, K, DIM_HEAD), lambda i: (i, 0, 0, 0)),
            pl.BlockSpec((1, HEADS, K, DIM_HEAD), lambda i: (i, 0, 0, 0)),
            pl.BlockSpec((C, inner), lambda i: (0, 0)),
            pl.BlockSpec((1, C), lambda i: (0, 0)),
            pl.BlockSpec((1, C), lambda i: (0, 0)),
            pl.BlockSpec((1, 1), lambda i: (0, 0)),
            pl.BlockSpec((1, L, C), lambda i: (i, 0, 0)),
        ],
        out_specs=pl.BlockSpec((1, L, C), lambda i: (i, 0, 0)),
        out_shape=jax.ShapeDtypeStruct((b, L, C), jnp.float32),
        compiler_params=arb(1),
    )(qh, ksel, vsel, woutT, go, bo, gam, qsT)

    return yT.transpose(0, 2, 1).reshape(b, C, D, H, W)
